# baseline (speedup 1.0000x reference)
import functools
import jax
import jax.numpy as jnp
import numpy as np
from jax import lax
from jax.experimental import pallas as pl
from jax.experimental.pallas import tpu as pltpu
from jax.experimental.pallas import tpu_sc as plsc

F = 128
RNG = 128
CH = 128
CHE = 80
NW = 32
NCORES = 2
BIG = 3.0e38
TM_NODE = 1024
TM_EDGE = 2000
TM_FIN = 512
TM_POSNEG = 2000

AVG_DEG_LOG = float(np.log(33.0))

_pc = pl.pallas_call
_pk = pl.kernel


def _multi_mm_body(x_ref, w_ref, b_ref, *o_refs):
    xv = x_ref[...]
    for t, o in enumerate(o_refs):
        o[...] = jnp.dot(xv, w_ref[t], preferred_element_type=jnp.float32) + b_ref[t]


def multi_mm(x, ws, bs, tm):
    M, K = x.shape
    k = ws.shape[0]
    grid = M // tm
    outs = [jax.ShapeDtypeStruct((M, F), jnp.float32)] * k
    return _pc(
        _multi_mm_body,
        grid=(grid,),
        in_specs=[pl.BlockSpec((tm, K), lambda i: (i, 0)),
                  pl.BlockSpec((k, K, F), lambda i: (0, 0, 0)),
                  pl.BlockSpec((k, 1, F), lambda i: (0, 0, 0))],
        out_specs=[pl.BlockSpec((tm, F), lambda i: (i, 0))] * k,
        out_shape=outs,
    )(x, ws, bs)


def mm_bias(x, w, b, tm):
    return multi_mm(x, w[None], b[None, None, :], tm)[0]


def _finalize_body(xp_ref, a_ref, u1_ref, u2_ref, mn_ref, mx_ref, dg_ref,
                   wp_ref, bp_ref, wl_ref, bl_ref, o_ref):
    A = a_ref[...]
    U1 = u1_ref[...]
    deg = jnp.sum(dg_ref[...], axis=1, keepdims=True)
    degc = jnp.maximum(deg, 1.0)
    inv = 1.0 / degc
    mean = (deg * A + U1) * inv
    mean2 = (deg * A * A + 2.0 * A * U1 + u2_ref[...]) * inv
    std = jnp.sqrt(jax.nn.relu(mean2 - mean * mean) + 1e-5)
    has = deg > 0.0
    mn = jnp.where(has, A + mn_ref[...], 0.0)
    mx = jnp.where(has, A + mx_ref[...], 0.0)
    amp = jnp.log(degc + 1.0) * (1.0 / AVG_DEG_LOG)
    ia = 1.0 / amp
    pieces = (xp_ref[...], mean, mn, mx, std,
              mean * amp, mn * amp, mx * amp, std * amp,
              mean * ia, mn * ia, mx * ia, std * ia)
    y = jnp.broadcast_to(bp_ref[...], pieces[0].shape)
    for t, pc in enumerate(pieces):
        y = y + jnp.dot(pc, wp_ref[t], preferred_element_type=jnp.float32)
    o_ref[...] = jnp.dot(y, wl_ref[...], preferred_element_type=jnp.float32) + bl_ref[...]


def finalize(xp, A, U1, U2, UMN, UMX, DEG, wpost, bpost, wlin, blin):
    tm = TM_FIN
    M = xp.shape[0]
    grid = M // tm
    nspec = pl.BlockSpec((tm, F), lambda i: (i, 0))
    return _pc(
        _finalize_body,
        grid=(grid,),
        in_specs=[nspec, nspec, nspec, nspec, nspec, nspec,
                  pl.BlockSpec((tm, F), lambda i: (i, 0)),
                  pl.BlockSpec((13, F, F), lambda i: (0, 0, 0)),
                  pl.BlockSpec((1, F), lambda i: (0, 0)),
                  pl.BlockSpec((F, F), lambda i: (0, 0)),
                  pl.BlockSpec((1, F), lambda i: (0, 0))],
        out_specs=nspec,
        out_shape=jax.ShapeDtypeStruct((M, F), jnp.float32),
    )(xp, A, U1, U2, UMN, UMX, DEG, wpost, bpost, wlin, blin)


def _bn_res_body(n_real, h_ref, xp_ref, g_ref, b_ref, o_ref):
    h = h_ref[...]
    mask = lax.broadcasted_iota(jnp.int32, h.shape, 0) < n_real
    hm = jnp.where(mask, h, 0.0)
    mu = jnp.sum(hm, axis=0, keepdims=True) * (1.0 / n_real)
    d = h - mu
    var = jnp.sum(jnp.where(mask, d * d, 0.0), axis=0, keepdims=True) * (1.0 / n_real)
    hn = g_ref[...] * d * jax.lax.rsqrt(var + 1e-5) + b_ref[...]
    o_ref[...] = (xp_ref[...] + jax.nn.relu(hn)) * 0.5


def bn_residual(h, xp, g, b, n_real):
    return _pc(
        functools.partial(_bn_res_body, n_real),
        out_shape=jax.ShapeDtypeStruct(h.shape, jnp.float32),
    )(h, xp, g[None, :], b[None, :])


def _emlp_fin_body(ea_ref, g_ref, w_ref, b_ref, o_ref):
    z = jnp.dot(g_ref[...], w_ref[...], preferred_element_type=jnp.float32) + b_ref[...]
    o_ref[...] = ea_ref[...] + 0.5 * z


def emlp_finish(ea, G, w1, b1):
    tm = TM_EDGE
    M = ea.shape[0]
    espec = pl.BlockSpec((tm, F), lambda i: (i, 0))
    return _pc(
        _emlp_fin_body,
        grid=(M // tm,),
        in_specs=[espec, espec,
                  pl.BlockSpec((F, F), lambda i: (0, 0)),
                  pl.BlockSpec((1, F), lambda i: (0, 0))],
        out_specs=espec,
        out_shape=jax.ShapeDtypeStruct((M, F), jnp.float32),
    )(ea, G, w1, b1[None, :])


def sc_aggregate(Bm, C, gath, eidg, dlg, desc, NR):
    mesh = plsc.VectorSubcoreMesh(core_axis_name="c", subcore_axis_name="s", num_cores=NCORES, num_subcores=NW // NCORES)
    rpw = (NR + NW - 1) // NW
    lanes = 16

    init0 = jnp.zeros((RNG + 1, F), jnp.float32)
    initmn = jnp.full((RNG + 1, F), BIG, jnp.float32)
    initmx = jnp.full((RNG + 1, F), -BIG, jnp.float32)
    initd = jnp.zeros((RNG + 1, F), jnp.float32)

    out_type = [jax.ShapeDtypeStruct((NR, RNG, F), jnp.float32)] * 4 + \
               [jax.ShapeDtypeStruct((NR, RNG, F), jnp.float32)]

    @functools.partial(
        _pk, mesh=mesh, out_type=out_type,
        compiler_params=pltpu.CompilerParams(needs_layout_passes=False),
        scratch_types=[
            pltpu.VMEM((16,), jnp.float32),
            pltpu.VMEM((CH,), jnp.int32),
            pltpu.VMEM((CH,), jnp.int32),
            pltpu.VMEM((CH,), jnp.int32),
            pltpu.VMEM((CH, F), jnp.float32),
            pltpu.VMEM((CH, F), jnp.float32),
            pltpu.VMEM((RNG + 1, F), jnp.float32),
            pltpu.VMEM((RNG + 1, F), jnp.float32),
            pltpu.VMEM((RNG + 1, F), jnp.float32),
            pltpu.VMEM((RNG + 1, F), jnp.float32),
            pltpu.VMEM((RNG + 1, F), jnp.float32),
            pltpu.SemaphoreType.DMA,
            pltpu.SemaphoreType.DMA,
        ],
    )
    def kern(b_hbm, c_hbm, g_hbm, e_hbm, dl_hbm, desc_hbm,
             i0_hbm, imn_hbm, imx_hbm, id_hbm,
             u1_hbm, u2_hbm, umn_hbm, umx_hbm, deg_hbm,
             descv, sidx, eidx, dlv, bg, cg, s1, s2, amn, amx, dga,
             sem1, sem2):
        cid = lax.axis_index("c")
        sid = lax.axis_index("s")
        wid = sid * NCORES + cid
        lane = lax.broadcasted_iota(jnp.int32, (lanes,), 0)
        ones = jnp.full((lanes,), 1.0, jnp.float32)

        def process(r):
            pltpu.sync_copy(desc_hbm.at[r], descv)
            v = descv[...]
            offc = v[0].astype(jnp.int32)
            nch = v[8].astype(jnp.int32)
            pltpu.sync_copy(i0_hbm, s1)
            pltpu.sync_copy(i0_hbm, s2)
            pltpu.sync_copy(imn_hbm, amn)
            pltpu.sync_copy(imx_hbm, amx)
            pltpu.sync_copy(id_hbm, dga)

            def chunk(i, carry):
                eoff = (offc + i) * CH
                pltpu.sync_copy(g_hbm.at[pl.ds(eoff, CH)], sidx)
                pltpu.sync_copy(e_hbm.at[pl.ds(eoff, CH)], eidx)
                pltpu.sync_copy(dl_hbm.at[pl.ds(eoff, CH)], dlv)
                cp1 = pltpu.async_copy(b_hbm.at[sidx], bg, sem1)
                cp2 = pltpu.async_copy(c_hbm.at[eidx], cg, sem2)
                cp1.wait()
                cp2.wait()
                for g in range(CH // lanes):
                    dl16 = dlv[pl.ds(g * lanes, lanes)]
                    plsc.addupdate_scatter(dga, [dl16, lane], ones)
                    row = lane + g * lanes
                    for j in range(F // lanes):
                        col = lane + j * lanes
                        bv = plsc.load_gather(bg, [row, col])
                        cv = plsc.load_gather(cg, [row, col])
                        u = bv + cv
                        plsc.addupdate_scatter(s1, [dl16, col], u)
                        plsc.addupdate_scatter(s2, [dl16, col], u * u)
                        mn = plsc.load_gather(amn, [dl16, col])
                        plsc.store_scatter(amn, [dl16, col], jnp.minimum(mn, u))
                        mx = plsc.load_gather(amx, [dl16, col])
                        plsc.store_scatter(amx, [dl16, col], jnp.maximum(mx, u))
                return carry

            lax.fori_loop(0, nch, chunk, 0)
            pltpu.sync_copy(s1.at[pl.ds(0, RNG)], u1_hbm.at[r])
            pltpu.sync_copy(s2.at[pl.ds(0, RNG)], u2_hbm.at[r])
            pltpu.sync_copy(amn.at[pl.ds(0, RNG)], umn_hbm.at[r])
            pltpu.sync_copy(amx.at[pl.ds(0, RNG)], umx_hbm.at[r])
            pltpu.sync_copy(dga.at[pl.ds(0, RNG)], deg_hbm.at[r])

        for rr in range(rpw):
            r = wid + rr * NW
            if (rr + 1) * NW <= NR:
                process(r)
            else:
                @pl.when(r < NR)
                def _():
                    process(r)

    return kern(Bm, C, gath, eidg, dlg, desc, init0, initmn, initmx, initd)


def sc_edge_gather(Q, R, P, srcs, dsts):
    E = P.shape[0]
    epw = E // NW
    nchunks = epw // CHE
    mesh = plsc.VectorSubcoreMesh(core_axis_name="c", subcore_axis_name="s", num_cores=NCORES, num_subcores=NW // NCORES)
    lanes = 16

    @functools.partial(
        _pk, mesh=mesh,
        out_type=jax.ShapeDtypeStruct((E, F), jnp.float32),
        compiler_params=pltpu.CompilerParams(needs_layout_passes=False),
        scratch_types=[
            pltpu.VMEM((CHE,), jnp.int32),
            pltpu.VMEM((CHE,), jnp.int32),
            pltpu.VMEM((CHE, F), jnp.float32),
            pltpu.VMEM((CHE, F), jnp.float32),
            pltpu.VMEM((CHE, F), jnp.float32),
            pltpu.VMEM((CHE, F), jnp.float32),
            pltpu.SemaphoreType.DMA,
            pltpu.SemaphoreType.DMA,
        ],
    )
    def kern(q_hbm, r_hbm, p_hbm, s_hbm, d_hbm, g_hbm,
             si, di, qg, rg, pg, go, sem1, sem2):
        cid = lax.axis_index("c")
        sid = lax.axis_index("s")
        wid = sid * NCORES + cid
        base = wid * epw

        def chunk(i, carry):
            eoff = base + i * CHE
            pltpu.sync_copy(s_hbm.at[pl.ds(eoff, CHE)], si)
            pltpu.sync_copy(d_hbm.at[pl.ds(eoff, CHE)], di)
            cp1 = pltpu.async_copy(q_hbm.at[si], qg, sem1)
            cp2 = pltpu.async_copy(r_hbm.at[di], rg, sem2)
            pltpu.sync_copy(p_hbm.at[pl.ds(eoff, CHE)], pg)
            cp1.wait()
            cp2.wait()

            def rowf(k, c2):
                for j in range(F // lanes):
                    sl = pl.ds(j * lanes, lanes)
                    go[k, sl] = jnp.maximum(pg[k, sl] + qg[k, sl] + rg[k, sl], 0.0)
                return c2

            lax.fori_loop(0, CHE, rowf, 0)
            pltpu.sync_copy(go, g_hbm.at[pl.ds(eoff, CHE)])
            return carry

        lax.fori_loop(0, nchunks, chunk, 0)

    return kern(Q, R, P, srcs, dsts)


def build_direction(aggr, other, n, NR):
    E = aggr.shape[0]
    order = jnp.argsort(aggr)
    a_s = aggr[order]
    o_s = other[order]
    rid = a_s // RNG
    dl = a_s - rid * RNG
    rp = jnp.searchsorted(a_s, jnp.arange(NR + 1, dtype=jnp.int32) * RNG).astype(jnp.int32)
    cnt = jnp.diff(rp)
    nch = (cnt + CH - 1) // CH
    offc = jnp.concatenate([jnp.zeros((1,), jnp.int32),
                            jnp.cumsum(nch).astype(jnp.int32)])[:NR]
    EP = E + NR * CH
    pos = offc[rid] * CH + (jnp.arange(E, dtype=jnp.int32) - rp[rid])
    gath = jnp.zeros((EP,), jnp.int32).at[pos].set(o_s)
    eidg = jnp.zeros((EP,), jnp.int32).at[pos].set(order.astype(jnp.int32))
    dlg = jnp.full((EP,), RNG, jnp.int32).at[pos].set(dl.astype(jnp.int32))
    lane0 = (jnp.arange(8) == 0).astype(jnp.float32)
    desc = jnp.concatenate([offc[:, None].astype(jnp.float32) * lane0[None, :],
                            nch[:, None].astype(jnp.float32) * lane0[None, :]],
                           axis=1)
    return dict(gath=gath, eidg=eidg, dlg=dlg, desc=desc)


def _pna_direction(xp, ea, p, d, NR):
    w1 = p['pre']['w'][:F]
    w2 = p['pre']['w'][F:2 * F]
    w3 = p['pre']['w'][2 * F:]
    zb = jnp.zeros((F,), jnp.float32)
    A, Bm = multi_mm(xp, jnp.stack([w1, w2]),
                     jnp.stack([zb[None, :], zb[None, :]]), TM_NODE)
    weff = mm_bias(p['edge_enc']['w'], w3, zb, 128)
    beff = mm_bias(p['edge_enc']['b'][None, :], w3, p['pre']['b'], 1)
    C = mm_bias(ea, weff, beff[0], TM_EDGE)
    U1, U2, UMN, UMX, DEG = sc_aggregate(Bm, C, d['gath'], d['eidg'], d['dlg'],
                                         d['desc'], NR)
    NPAD = NR * RNG
    return finalize(xp, A,
                    U1.reshape(NPAD, F), U2.reshape(NPAD, F),
                    UMN.reshape(NPAD, F), UMX.reshape(NPAD, F),
                    DEG.reshape(NPAD, F),
                    p['post']['w'].reshape(13, F, F), p['post']['b'][None, :],
                    p['lin']['w'], p['lin']['b'][None, :])


def kernel(x, edge_index, edge_attr, pos_edge_index, pos_edge_attr,
           neg_edge_index, neg_edge_attr, params):
    n = x.shape[0]
    NR = ((((n + RNG - 1) // RNG) + 1) // 2) * 2
    NPAD = NR * RNG
    src = edge_index[0].astype(jnp.int32)
    dst = edge_index[1].astype(jnp.int32)
    d_f = build_direction(dst, src, n, NR)
    d_b = build_direction(src, dst, n, NR)

    xpad = jnp.concatenate([x, jnp.zeros((NPAD - n, x.shape[1]), jnp.float32)], 0)
    xp = mm_bias(xpad, params['node_emb']['w'], params['node_emb']['b'], TM_NODE)
    we, be = params['edge_emb']['w'], params['edge_emb']['b']
    ea = mm_bias(edge_attr, we, be, TM_EDGE)
    pe = mm_bias(pos_edge_attr, we, be, TM_POSNEG)
    ne = mm_bias(neg_edge_attr, we, be, TM_POSNEG)

    for lp in params['layers']:
        a_in = _pna_direction(xp, ea, lp['forw'], d_f, NR)
        a_out = _pna_direction(xp, ea, lp['back'], d_b, NR)
        h = mm_bias(jnp.concatenate([xp, a_in, a_out], axis=1),
                    lp['hlin']['w'], lp['hlin']['b'], TM_NODE)
        xp = bn_residual(h, xp, lp['bn_g'], lp['bn_b'], n)
        w0 = lp['emlp0']['w']
        zb = jnp.zeros((F,), jnp.float32)
        Q, R = multi_mm(xp, jnp.stack([w0[:F], w0[F:2 * F]]),
                        jnp.stack([zb[None, :], zb[None, :]]), TM_NODE)
        P = mm_bias(ea, w0[2 * F:], lp['emlp0']['b'], TM_EDGE)
        G = sc_edge_gather(Q, R, P, src, dst)
        ea = emlp_finish(ea, G, lp['emlp1']['w'], lp['emlp1']['b'])

    return xp[:n], pe, ne


# speedup vs baseline: 4.5647x; 4.5647x over previous
import functools
import jax
import jax.numpy as jnp
import numpy as np
from jax import lax
from jax.experimental import pallas as pl
from jax.experimental.pallas import tpu as pltpu
from jax.experimental.pallas import tpu_sc as plsc

F = 128
RNG = 128
CH = 128
CHE = 80
NW = 32
NCORES = 2
BIG = 3.0e38
TM_NODE = 1024
TM_EDGE = 2000
TM_FIN = 512
TM_POSNEG = 2000

AVG_DEG_LOG = float(np.log(33.0))

_pc = pl.pallas_call
_pk = pl.kernel


def _multi_mm_body(x_ref, w_ref, b_ref, *o_refs):
    xv = x_ref[...]
    for t, o in enumerate(o_refs):
        o[...] = jnp.dot(xv, w_ref[t], preferred_element_type=jnp.float32) + b_ref[t]


def multi_mm(x, ws, bs, tm):
    M, K = x.shape
    k = ws.shape[0]
    grid = M // tm
    outs = [jax.ShapeDtypeStruct((M, F), jnp.float32)] * k
    return _pc(
        _multi_mm_body,
        grid=(grid,),
        in_specs=[pl.BlockSpec((tm, K), lambda i: (i, 0)),
                  pl.BlockSpec((k, K, F), lambda i: (0, 0, 0)),
                  pl.BlockSpec((k, 1, F), lambda i: (0, 0, 0))],
        out_specs=[pl.BlockSpec((tm, F), lambda i: (i, 0))] * k,
        out_shape=outs,
    )(x, ws, bs)


def mm_bias(x, w, b, tm):
    return multi_mm(x, w[None], b[None, None, :], tm)[0]


def _finalize_body(xp_ref, a_ref, u1_ref, u2_ref, mn_ref, mx_ref, dg_ref,
                   wp_ref, bp_ref, wl_ref, bl_ref, o_ref):
    A = a_ref[...]
    U1 = u1_ref[...]
    deg = jnp.sum(dg_ref[...], axis=1, keepdims=True)
    degc = jnp.maximum(deg, 1.0)
    inv = 1.0 / degc
    mean = (deg * A + U1) * inv
    mean2 = (deg * A * A + 2.0 * A * U1 + u2_ref[...]) * inv
    std = jnp.sqrt(jax.nn.relu(mean2 - mean * mean) + 1e-5)
    has = deg > 0.0
    mn = jnp.where(has, A + mn_ref[...], 0.0)
    mx = jnp.where(has, A + mx_ref[...], 0.0)
    amp = jnp.log(degc + 1.0) * (1.0 / AVG_DEG_LOG)
    ia = 1.0 / amp
    pieces = (xp_ref[...], mean, mn, mx, std,
              mean * amp, mn * amp, mx * amp, std * amp,
              mean * ia, mn * ia, mx * ia, std * ia)
    cat = jnp.concatenate(pieces, axis=1)
    y = jnp.dot(cat, wp_ref[...], preferred_element_type=jnp.float32) + bp_ref[...]
    o_ref[...] = jnp.dot(y, wl_ref[...], preferred_element_type=jnp.float32) + bl_ref[...]


def finalize(xp, A, U1, U2, UMN, UMX, DEG, wpost, bpost, wlin, blin):
    tm = TM_FIN
    M = xp.shape[0]
    grid = M // tm
    nspec = pl.BlockSpec((tm, F), lambda i: (i, 0))
    return _pc(
        _finalize_body,
        grid=(grid,),
        in_specs=[nspec, nspec, nspec, nspec, nspec, nspec,
                  pl.BlockSpec((tm, F), lambda i: (i, 0)),
                  pl.BlockSpec((13 * F, F), lambda i: (0, 0)),
                  pl.BlockSpec((1, F), lambda i: (0, 0)),
                  pl.BlockSpec((F, F), lambda i: (0, 0)),
                  pl.BlockSpec((1, F), lambda i: (0, 0))],
        out_specs=nspec,
        out_shape=jax.ShapeDtypeStruct((M, F), jnp.float32),
    )(xp, A, U1, U2, UMN, UMX, DEG, wpost, bpost, wlin, blin)


def _bn_res_body(n_real, h_ref, xp_ref, g_ref, b_ref, o_ref):
    h = h_ref[...]
    mask = lax.broadcasted_iota(jnp.int32, h.shape, 0) < n_real
    hm = jnp.where(mask, h, 0.0)
    mu = jnp.sum(hm, axis=0, keepdims=True) * (1.0 / n_real)
    d = h - mu
    var = jnp.sum(jnp.where(mask, d * d, 0.0), axis=0, keepdims=True) * (1.0 / n_real)
    hn = g_ref[...] * d * jax.lax.rsqrt(var + 1e-5) + b_ref[...]
    o_ref[...] = (xp_ref[...] + jax.nn.relu(hn)) * 0.5


def bn_residual(h, xp, g, b, n_real):
    return _pc(
        functools.partial(_bn_res_body, n_real),
        out_shape=jax.ShapeDtypeStruct(h.shape, jnp.float32),
    )(h, xp, g[None, :], b[None, :])


def _emlp_fin_body(ea_ref, g_ref, w_ref, b_ref, o_ref):
    z = jnp.dot(g_ref[...], w_ref[...], preferred_element_type=jnp.float32) + b_ref[...]
    o_ref[...] = ea_ref[...] + 0.5 * z


def emlp_finish(ea, G, w1, b1):
    tm = TM_EDGE
    M = ea.shape[0]
    espec = pl.BlockSpec((tm, F), lambda i: (i, 0))
    return _pc(
        _emlp_fin_body,
        grid=(M // tm,),
        in_specs=[espec, espec,
                  pl.BlockSpec((F, F), lambda i: (0, 0)),
                  pl.BlockSpec((1, F), lambda i: (0, 0))],
        out_specs=espec,
        out_shape=jax.ShapeDtypeStruct((M, F), jnp.float32),
    )(ea, G, w1, b1[None, :])


def sc_aggregate(Bm, C, order_p, as_p, other, desc, NR):
    mesh = plsc.VectorSubcoreMesh(core_axis_name="c", subcore_axis_name="s", num_cores=NCORES, num_subcores=NW // NCORES)
    rpw = (NR + NW - 1) // NW
    lanes = 16

    init0 = jnp.zeros((RNG + 1, F), jnp.float32)
    initmn = jnp.full((RNG + 1, F), BIG, jnp.float32)
    initmx = jnp.full((RNG + 1, F), -BIG, jnp.float32)
    initd = jnp.zeros((RNG + 1, F), jnp.float32)

    out_type = [jax.ShapeDtypeStruct((NR, RNG, F), jnp.float32)] * 5

    @functools.partial(
        _pk, mesh=mesh, out_type=out_type,
        compiler_params=pltpu.CompilerParams(needs_layout_passes=False),
        scratch_types=[
            pltpu.VMEM((16,), jnp.float32),
            pltpu.VMEM((CH,), jnp.int32),
            pltpu.VMEM((CH,), jnp.int32),
            pltpu.VMEM((CH,), jnp.int32),
            pltpu.VMEM((CH, F), jnp.float32),
            pltpu.VMEM((CH, F), jnp.float32),
            pltpu.VMEM((RNG + 1, F), jnp.float32),
            pltpu.VMEM((RNG + 1, F), jnp.float32),
            pltpu.VMEM((RNG + 1, F), jnp.float32),
            pltpu.VMEM((RNG + 1, F), jnp.float32),
            pltpu.VMEM((RNG + 1, F), jnp.float32),
            pltpu.SemaphoreType.DMA,
            pltpu.SemaphoreType.DMA,
        ],
    )
    def kern(b_hbm, c_hbm, ord_hbm, as_hbm, oth_hbm, desc_hbm,
             i0_hbm, imn_hbm, imx_hbm, id_hbm,
             u1_hbm, u2_hbm, umn_hbm, umx_hbm, deg_hbm,
             descv, eidx, av, ov, bg, cg, s1, s2, amn, amx, dga,
             sem1, sem2):
        cid = lax.axis_index("c")
        sid = lax.axis_index("s")
        wid = sid * NCORES + cid
        lane = lax.broadcasted_iota(jnp.int32, (lanes,), 0)
        ones = jnp.full((lanes,), 1.0, jnp.float32)

        def process(r):
            pltpu.sync_copy(desc_hbm.at[r], descv)
            v = descv[...]
            start = v[0].astype(jnp.int32)
            lo = v[4].astype(jnp.int32)
            hi = v[8].astype(jnp.int32)
            nch = v[12].astype(jnp.int32)
            rbase = r * RNG
            pltpu.sync_copy(i0_hbm, s1)
            pltpu.sync_copy(i0_hbm, s2)
            pltpu.sync_copy(imn_hbm, amn)
            pltpu.sync_copy(imx_hbm, amx)
            pltpu.sync_copy(id_hbm, dga)

            def chunk(i, carry):
                base = start + i * CH
                abase = pl.multiple_of(base, 8)
                pltpu.sync_copy(ord_hbm.at[pl.ds(abase, CH)], eidx)
                pltpu.sync_copy(as_hbm.at[pl.ds(abase, CH)], av)
                cpo = pltpu.async_copy(oth_hbm.at[eidx], ov, sem1)
                cpc = pltpu.async_copy(c_hbm.at[eidx], cg, sem2)
                cpo.wait()
                cpb = pltpu.async_copy(b_hbm.at[ov], bg, sem1)
                cpc.wait()
                cpb.wait()
                for g in range(CH // lanes):
                    k16 = base + g * lanes + lane
                    avv = av[pl.ds(g * lanes, lanes)]
                    valid = (k16 >= lo) & (k16 < hi)
                    dl16 = jnp.where(valid, avv - rbase, RNG)
                    plsc.addupdate_scatter(dga, [dl16, lane], ones)
                    row = lane + g * lanes
                    for j in range(F // lanes):
                        col = lane + j * lanes
                        bv = plsc.load_gather(bg, [row, col])
                        cv = plsc.load_gather(cg, [row, col])
                        u = bv + cv
                        plsc.addupdate_scatter(s1, [dl16, col], u)
                        plsc.addupdate_scatter(s2, [dl16, col], u * u)
                        mn = plsc.load_gather(amn, [dl16, col])
                        plsc.store_scatter(amn, [dl16, col], jnp.minimum(mn, u))
                        mx = plsc.load_gather(amx, [dl16, col])
                        plsc.store_scatter(amx, [dl16, col], jnp.maximum(mx, u))
                return carry

            lax.fori_loop(0, nch, chunk, 0)
            pltpu.sync_copy(s1.at[pl.ds(0, RNG)], u1_hbm.at[r])
            pltpu.sync_copy(s2.at[pl.ds(0, RNG)], u2_hbm.at[r])
            pltpu.sync_copy(amn.at[pl.ds(0, RNG)], umn_hbm.at[r])
            pltpu.sync_copy(amx.at[pl.ds(0, RNG)], umx_hbm.at[r])
            pltpu.sync_copy(dga.at[pl.ds(0, RNG)], deg_hbm.at[r])

        for rr in range(rpw):
            r = wid + rr * NW
            if (rr + 1) * NW <= NR:
                process(r)
            else:
                @pl.when(r < NR)
                def _():
                    process(r)

    return kern(Bm, C, order_p, as_p, other, desc, init0, initmn, initmx, initd)


def sc_edge_gather(Q, R, P, srcs, dsts):
    E = P.shape[0]
    epw = E // NW
    nchunks = epw // CHE
    mesh = plsc.VectorSubcoreMesh(core_axis_name="c", subcore_axis_name="s", num_cores=NCORES, num_subcores=NW // NCORES)
    lanes = 16

    @functools.partial(
        _pk, mesh=mesh,
        out_type=jax.ShapeDtypeStruct((E, F), jnp.float32),
        compiler_params=pltpu.CompilerParams(needs_layout_passes=False),
        scratch_types=[
            pltpu.VMEM((CHE,), jnp.int32),
            pltpu.VMEM((CHE,), jnp.int32),
            pltpu.VMEM((CHE, F), jnp.float32),
            pltpu.VMEM((CHE, F), jnp.float32),
            pltpu.VMEM((CHE, F), jnp.float32),
            pltpu.VMEM((CHE, F), jnp.float32),
            pltpu.SemaphoreType.DMA,
            pltpu.SemaphoreType.DMA,
        ],
    )
    def kern(q_hbm, r_hbm, p_hbm, s_hbm, d_hbm, g_hbm,
             si, di, qg, rg, pg, go, sem1, sem2):
        cid = lax.axis_index("c")
        sid = lax.axis_index("s")
        wid = sid * NCORES + cid
        base = wid * epw

        def chunk(i, carry):
            eoff = base + i * CHE
            pltpu.sync_copy(s_hbm.at[pl.ds(eoff, CHE)], si)
            pltpu.sync_copy(d_hbm.at[pl.ds(eoff, CHE)], di)
            cp1 = pltpu.async_copy(q_hbm.at[si], qg, sem1)
            cp2 = pltpu.async_copy(r_hbm.at[di], rg, sem2)
            pltpu.sync_copy(p_hbm.at[pl.ds(eoff, CHE)], pg)
            cp1.wait()
            cp2.wait()

            def rowf(k, c2):
                for j in range(F // lanes):
                    sl = pl.ds(j * lanes, lanes)
                    go[k, sl] = jnp.maximum(pg[k, sl] + qg[k, sl] + rg[k, sl], 0.0)
                return c2

            lax.fori_loop(0, CHE, rowf, 0)
            pltpu.sync_copy(go, g_hbm.at[pl.ds(eoff, CHE)])
            return carry

        lax.fori_loop(0, nchunks, chunk, 0)

    return kern(Q, R, P, srcs, dsts)


def build_direction(aggr, other, n, NR):
    E = aggr.shape[0]
    a_s, order = lax.sort_key_val(aggr, jnp.arange(E, dtype=jnp.int32))
    rp = jnp.searchsorted(a_s, jnp.arange(NR + 1, dtype=jnp.int32) * RNG).astype(jnp.int32)
    lo = rp[:-1]
    hi = rp[1:]
    start = (lo // 8) * 8
    nch = (hi - start + CH - 1) // CH
    pad = jnp.zeros((CH,), jnp.int32)
    order_p = jnp.concatenate([order, pad])
    as_p = jnp.concatenate([a_s, pad])
    oh = lambda k: (jnp.arange(16) == k).astype(jnp.float32)[None, :]
    desc = (start[:, None].astype(jnp.float32) * oh(0)
            + lo[:, None].astype(jnp.float32) * oh(4)
            + hi[:, None].astype(jnp.float32) * oh(8)
            + nch[:, None].astype(jnp.float32) * oh(12))
    return dict(order=order_p, a_s=as_p, desc=desc, other=other)


def _pna_direction(xp, ea, p, d, NR):
    w1 = p['pre']['w'][:F]
    w2 = p['pre']['w'][F:2 * F]
    w3 = p['pre']['w'][2 * F:]
    zb = jnp.zeros((F,), jnp.float32)
    A, Bm = multi_mm(xp, jnp.stack([w1, w2]),
                     jnp.stack([zb[None, :], zb[None, :]]), TM_NODE)
    weff = mm_bias(p['edge_enc']['w'], w3, zb, 128)
    beff = mm_bias(p['edge_enc']['b'][None, :], w3, p['pre']['b'], 1)
    C = mm_bias(ea, weff, beff[0], TM_EDGE)
    U1, U2, UMN, UMX, DEG = sc_aggregate(Bm, C, d['order'], d['a_s'],
                                         d['other'], d['desc'], NR)
    NPAD = NR * RNG
    return finalize(xp, A,
                    U1.reshape(NPAD, F), U2.reshape(NPAD, F),
                    UMN.reshape(NPAD, F), UMX.reshape(NPAD, F),
                    DEG.reshape(NPAD, F),
                    p['post']['w'], p['post']['b'][None, :],
                    p['lin']['w'], p['lin']['b'][None, :])


def kernel(x, edge_index, edge_attr, pos_edge_index, pos_edge_attr,
           neg_edge_index, neg_edge_attr, params):
    n = x.shape[0]
    NR = ((((n + RNG - 1) // RNG) + 1) // 2) * 2
    NPAD = NR * RNG
    src = edge_index[0].astype(jnp.int32)
    dst = edge_index[1].astype(jnp.int32)
    d_f = build_direction(dst, src, n, NR)
    d_b = build_direction(src, dst, n, NR)

    xpad = jnp.concatenate([x, jnp.zeros((NPAD - n, x.shape[1]), jnp.float32)], 0)
    xp = mm_bias(xpad, params['node_emb']['w'], params['node_emb']['b'], TM_NODE)
    we, be = params['edge_emb']['w'], params['edge_emb']['b']
    ea = mm_bias(edge_attr, we, be, TM_EDGE)
    pe = mm_bias(pos_edge_attr, we, be, TM_POSNEG)
    ne = mm_bias(neg_edge_attr, we, be, TM_POSNEG)

    for lp in params['layers']:
        a_in = _pna_direction(xp, ea, lp['forw'], d_f, NR)
        a_out = _pna_direction(xp, ea, lp['back'], d_b, NR)
        h = mm_bias(jnp.concatenate([xp, a_in, a_out], axis=1),
                    lp['hlin']['w'], lp['hlin']['b'], TM_NODE)
        xp = bn_residual(h, xp, lp['bn_g'], lp['bn_b'], n)
        w0 = lp['emlp0']['w']
        zb = jnp.zeros((F,), jnp.float32)
        Q, R = multi_mm(xp, jnp.stack([w0[:F], w0[F:2 * F]]),
                        jnp.stack([zb[None, :], zb[None, :]]), TM_NODE)
        P = mm_bias(ea, w0[2 * F:], lp['emlp0']['b'], TM_EDGE)
        G = sc_edge_gather(Q, R, P, src, dst)
        ea = emlp_finish(ea, G, lp['emlp1']['w'], lp['emlp1']['b'])

    return xp[:n], pe, ne
